# Initial kernel scaffold; baseline (speedup 1.0000x reference)
import jax
import jax.numpy as jnp
from jax.experimental import pallas as pl


def kernel(x, edge_attr, edge_index, batch, W0, b0, Wl, bl, Wr, We, be, Wih, Whh, bih, bhh, Wp1, bp1, Wp2, bp2):
    raise NotImplementedError("write your pallas kernel here")


# speedup vs baseline: 1.1607x; 1.1607x over previous
import functools

import jax
import jax.numpy as jnp
from jax import lax
from jax.experimental import pallas as pl
from jax.experimental.pallas import tpu as pltpu

_MAX_DEG = 10
_NUM_GRAPHS = 16
_BLK = 512


def _tc_a_body(x_ref, ea_ref, w0t_ref, b0_ref, wa_ref, wb_ref,
               out0_ref, a_ref, b_ref):
    x = x_ref[...]
    ea = ea_ref[...]
    out0_ref[...] = jax.nn.relu(
        jnp.dot(x, w0t_ref[...], preferred_element_type=jnp.float32)
        + b0_ref[...])
    a_ref[...] = jnp.dot(ea, wa_ref[...], preferred_element_type=jnp.float32)
    b_ref[...] = jnp.dot(ea, wb_ref[...], preferred_element_type=jnp.float32)


def _tc_a(xp, eap, W0T, b0, WA, WB, n_pad):
    grid = n_pad // _BLK
    blk = lambda i: (i, 0)
    full = lambda i: (0, 0)
    return pl.pallas_call(
        _tc_a_body,
        grid=(grid,),
        in_specs=[
            pl.BlockSpec((_BLK, 128), blk),
            pl.BlockSpec((_BLK, 128), blk),
            pl.BlockSpec((128, 128), full),
            pl.BlockSpec((1, 128), full),
            pl.BlockSpec((128, 128), full),
            pl.BlockSpec((128, 128), full),
        ],
        out_specs=[
            pl.BlockSpec((_BLK, 128), blk),
            pl.BlockSpec((_BLK, 128), blk),
            pl.BlockSpec((_BLK, 128), blk),
        ],
        out_shape=[jax.ShapeDtypeStruct((n_pad, 128), jnp.float32)] * 3,
    )(xp, eap, W0T, b0, WA, WB)


def _tc_b_body(hsum_ref, prev_ref, cnt_ref, a_ref, m_ref,
               wlt_ref, wrt_ref, bl_ref, wiht_ref, whht_ref,
               bih_ref, bhh_ref, be_ref, wa_ref, wb_ref,
               h_ref, mfo_ref, agg_ref, an_ref, bn_ref):
    hsum = hsum_ref[...]
    prev = prev_ref[...]
    hall = jnp.dot(hsum, wlt_ref[...], preferred_element_type=jnp.float32)
    xall = jnp.dot(prev, wrt_ref[...], preferred_element_type=jnp.float32)
    allr = (hall + xall).reshape(_BLK, _MAX_DEG + 1, 128) + bl_ref[...]
    cnt = cnt_ref[...]
    deg = jnp.clip(cnt, 0.0, float(_MAX_DEG)).reshape(_BLK, 1, 1).astype(
        jnp.int32)
    sel = lax.broadcasted_iota(jnp.int32, (_BLK, _MAX_DEG + 1, 1), 1)
    m = jnp.sum(jnp.where(sel == deg, allr, 0.0), axis=1)
    m = jax.nn.relu(m)
    mfo_ref[...] = m
    gi = jnp.dot(m, wiht_ref[...], preferred_element_type=jnp.float32) \
        + bih_ref[...]
    gh = jnp.dot(prev, whht_ref[...], preferred_element_type=jnp.float32) \
        + bhh_ref[...]
    ir, iz, inn = gi[:, :128], gi[:, 128:256], gi[:, 256:]
    hr, hz, hn = gh[:, :128], gh[:, 128:256], gh[:, 256:]
    r = jax.nn.sigmoid(ir + hr)
    z = jax.nn.sigmoid(iz + hz)
    n = jnp.tanh(inn + r * hn)
    h_ref[...] = (1.0 - z) * n + z * prev
    agg = jnp.where(cnt > 0.0, a_ref[...] + be_ref[...] + m_ref[...], 0.0)
    agg_ref[...] = agg
    ea = jax.nn.relu(agg)
    an_ref[...] = jnp.dot(ea, wa_ref[...], preferred_element_type=jnp.float32)
    bn_ref[...] = jnp.dot(ea, wb_ref[...], preferred_element_type=jnp.float32)


def _tc_b(hsum, prev, cnt, A, M, WlT, WrT, bl3, WihT, WhhT, bih, bhh, be,
          WA, WB, n_pad):
    grid = n_pad // _BLK
    blk = lambda i: (i, 0)
    full = lambda i: (0, 0)
    return pl.pallas_call(
        _tc_b_body,
        grid=(grid,),
        in_specs=[
            pl.BlockSpec((_BLK, 128), blk),
            pl.BlockSpec((_BLK, 128), blk),
            pl.BlockSpec((_BLK, 1), blk),
            pl.BlockSpec((_BLK, 128), blk),
            pl.BlockSpec((_BLK, 128), blk),
            pl.BlockSpec((128, (_MAX_DEG + 1) * 128), full),
            pl.BlockSpec((128, (_MAX_DEG + 1) * 128), full),
            pl.BlockSpec((1, _MAX_DEG + 1, 128), lambda i: (0, 0, 0)),
            pl.BlockSpec((128, 384), full),
            pl.BlockSpec((128, 384), full),
            pl.BlockSpec((1, 384), full),
            pl.BlockSpec((1, 384), full),
            pl.BlockSpec((1, 128), full),
            pl.BlockSpec((128, 128), full),
            pl.BlockSpec((128, 128), full),
        ],
        out_specs=[pl.BlockSpec((_BLK, 128), blk)] * 5,
        out_shape=[jax.ShapeDtypeStruct((n_pad, 128), jnp.float32)] * 5,
    )(hsum, prev, cnt, A, M, WlT, WrT, bl3, WihT, WhhT, bih, bhh, be, WA, WB)


def _tc_d_body(s1_ref, agg2_ref, cnt_ref, oh_ref, wp1t_ref, bp1_ref,
               wp2t_ref, bp2_ref, g_ref, acc_ref):
    i = pl.program_id(0)

    @pl.when(i == 0)
    def _():
        acc_ref[...] = jnp.zeros_like(acc_ref)

    s2 = jax.nn.relu(agg2_ref[...]) * cnt_ref[...]
    oh = oh_ref[...]
    p1 = jnp.dot(oh.T, s1_ref[...], preferred_element_type=jnp.float32)
    p2 = jnp.dot(oh.T, s2, preferred_element_type=jnp.float32)
    acc_ref[...] += jnp.concatenate([p1, p2], axis=1)

    @pl.when(i == pl.num_programs(0) - 1)
    def _():
        gsum = acc_ref[...]
        hid = jax.nn.relu(
            jnp.dot(gsum, wp1t_ref[...], preferred_element_type=jnp.float32)
            + bp1_ref[...])
        g_ref[...] = jnp.dot(hid, wp2t_ref[...],
                             preferred_element_type=jnp.float32) + bp2_ref[...]


def _tc_d(s1, agg2, cnt, oh, Wp1T, bp1, Wp2T, bp2, n_pad):
    grid = n_pad // _BLK
    blk = lambda i: (i, 0)
    full = lambda i: (0, 0)
    return pl.pallas_call(
        _tc_d_body,
        grid=(grid,),
        in_specs=[
            pl.BlockSpec((_BLK, 128), blk),
            pl.BlockSpec((_BLK, 128), blk),
            pl.BlockSpec((_BLK, 1), blk),
            pl.BlockSpec((_BLK, 16), blk),
            pl.BlockSpec((256, 32), full),
            pl.BlockSpec((1, 32), full),
            pl.BlockSpec((32, 1), full),
            pl.BlockSpec((1, 1), full),
        ],
        out_specs=pl.BlockSpec((16, 1), full),
        out_shape=jax.ShapeDtypeStruct((16, 1), jnp.float32),
        scratch_shapes=[pltpu.VMEM((16, 256), jnp.float32)],
        compiler_params=pltpu.CompilerParams(
            dimension_semantics=("arbitrary",)),
    )(s1, agg2, cnt, oh, Wp1T, bp1, Wp2T, bp2)


def _emb_edge_body(agg2_ref, out_ref):
    i = pl.program_id(0)
    out_ref[...] = jnp.where(i < 5, agg2_ref[...], 0.0)


def _emb_edge(agg2, E):
    blk_rows = 2000
    grid = E // blk_rows
    return pl.pallas_call(
        _emb_edge_body,
        grid=(grid,),
        in_specs=[pl.BlockSpec((blk_rows, 128),
                               lambda i: (jnp.minimum(i, 4), 0))],
        out_specs=pl.BlockSpec((blk_rows, 128), lambda i: (i, 0)),
        out_shape=jax.ShapeDtypeStruct((E, 128), jnp.float32),
    )(agg2)


def kernel(x, edge_attr, edge_index, batch, W0, b0, Wl, bl, Wr, We, be,
           Wih, Whh, bih, bhh, Wp1, bp1, Wp2, bp2):
    N, D = x.shape
    E = edge_attr.shape[0]
    n_pad = ((N + _BLK - 1) // _BLK) * _BLK
    row, col = edge_index[0], edge_index[1]

    ED = We.shape[0]
    W0T = W0.T
    WA = (We[:, :ED] - We[:, ED:]).T
    WB = We[:, ED:].T
    WlT = jnp.transpose(Wl, (2, 0, 1)).reshape(D, (_MAX_DEG + 1) * D)
    WrT = jnp.transpose(Wr, (2, 0, 1)).reshape(D, (_MAX_DEG + 1) * D)
    bl3 = bl.reshape(1, _MAX_DEG + 1, D)
    WihT, WhhT = Wih.T, Whh.T
    Wp1T, Wp2T = Wp1.T, Wp2.T
    b0r = b0.reshape(1, D)
    ber = be.reshape(1, D)
    bihr, bhhr = bih.reshape(1, 3 * D), bhh.reshape(1, 3 * D)
    bp1r, bp2r = bp1.reshape(1, 32), bp2.reshape(1, 1)

    pad = n_pad - N
    xp = jnp.pad(x, ((0, pad), (0, 0)))
    eap = jnp.pad(edge_attr[:N], ((0, pad), (0, 0)))
    ohp = jnp.pad(
        (batch[:, None] == jnp.arange(_NUM_GRAPHS)[None, :]).astype(
            jnp.float32), ((0, pad), (0, 0)))

    out0, A1, B1 = _tc_a(xp, eap, W0T, b0r, WA, WB, n_pad)

    cnt = jnp.bincount(col, length=n_pad).astype(jnp.float32)[:, None]
    hsum1 = jax.ops.segment_sum(out0[row], col, num_segments=n_pad)
    M1 = jax.ops.segment_max(B1[row], col, num_segments=n_pad)
    M1 = jnp.where(jnp.isfinite(M1), M1, 0.0)

    h1, _m1, _agg1, A2, B2 = _tc_b(hsum1, out0, cnt, A1, M1, WlT, WrT, bl3,
                                   WihT, WhhT, bihr, bhhr, ber, WA, WB, n_pad)

    hsum2 = jax.ops.segment_sum(h1[row], col, num_segments=n_pad)
    M2 = jax.ops.segment_max(B2[row], col, num_segments=n_pad)
    M2 = jnp.where(jnp.isfinite(M2), M2, 0.0)

    h2, m2, agg2, _A3, _B3 = _tc_b(hsum2, h1, cnt, A2, M2, WlT, WrT, bl3,
                                   WihT, WhhT, bihr, bhhr, ber, WA, WB, n_pad)

    s1 = jax.ops.segment_sum(h2[row], col, num_segments=n_pad)

    g = _tc_d(s1, agg2, cnt, ohp, Wp1T, bp1r, Wp2T, bp2r, n_pad)

    emb_node = m2[:N]
    emb_edge = _emb_edge(agg2[:N], E)
    return (g, emb_node, emb_edge)


# speedup vs baseline: 1.7920x; 1.5439x over previous
import functools

import jax
import jax.numpy as jnp
from jax import lax
from jax.experimental import pallas as pl
from jax.experimental.pallas import tpu as pltpu
from jax.experimental.pallas import tpu_sc as plsc

_MAX_DEG = 10
_NUM_GRAPHS = 16
_BLK = 512


def _tc_a_body(x_ref, ea_ref, w0t_ref, b0_ref, wa_ref, wb_ref,
               out0_ref, a_ref, b_ref):
    x = x_ref[...]
    ea = ea_ref[...]
    out0_ref[...] = jax.nn.relu(
        jnp.dot(x, w0t_ref[...], preferred_element_type=jnp.float32)
        + b0_ref[...])
    a_ref[...] = jnp.dot(ea, wa_ref[...], preferred_element_type=jnp.float32)
    b_ref[...] = jnp.dot(ea, wb_ref[...], preferred_element_type=jnp.float32)


def _tc_a(xp, eap, W0T, b0, WA, WB, n_pad):
    grid = n_pad // _BLK
    blk = lambda i: (i, 0)
    full = lambda i: (0, 0)
    return pl.pallas_call(
        _tc_a_body,
        grid=(grid,),
        in_specs=[
            pl.BlockSpec((_BLK, 128), blk),
            pl.BlockSpec((_BLK, 128), blk),
            pl.BlockSpec((128, 128), full),
            pl.BlockSpec((1, 128), full),
            pl.BlockSpec((128, 128), full),
            pl.BlockSpec((128, 128), full),
        ],
        out_specs=[
            pl.BlockSpec((_BLK, 128), blk),
            pl.BlockSpec((_BLK, 128), blk),
            pl.BlockSpec((_BLK, 128), blk),
        ],
        out_shape=[jax.ShapeDtypeStruct((n_pad, 128), jnp.float32)] * 3,
    )(xp, eap, W0T, b0, WA, WB)


def _tc_b_body(hsum_ref, prev_ref, cnt_ref, a_ref, m_ref,
               wlt_ref, wrt_ref, bl_ref, wiht_ref, whht_ref,
               bih_ref, bhh_ref, be_ref, wa_ref, wb_ref,
               h_ref, mfo_ref, agg_ref, an_ref, bn_ref):
    hsum = hsum_ref[...]
    prev = prev_ref[...]
    hall = jnp.dot(hsum, wlt_ref[...], preferred_element_type=jnp.float32)
    xall = jnp.dot(prev, wrt_ref[...], preferred_element_type=jnp.float32)
    allr = (hall + xall).reshape(_BLK, _MAX_DEG + 1, 128) + bl_ref[...]
    cnt = cnt_ref[...]
    deg = jnp.clip(cnt, 0.0, float(_MAX_DEG)).reshape(_BLK, 1, 1).astype(
        jnp.int32)
    sel = lax.broadcasted_iota(jnp.int32, (_BLK, _MAX_DEG + 1, 1), 1)
    m = jnp.sum(jnp.where(sel == deg, allr, 0.0), axis=1)
    m = jax.nn.relu(m)
    mfo_ref[...] = m
    gi = jnp.dot(m, wiht_ref[...], preferred_element_type=jnp.float32) \
        + bih_ref[...]
    gh = jnp.dot(prev, whht_ref[...], preferred_element_type=jnp.float32) \
        + bhh_ref[...]
    ir, iz, inn = gi[:, :128], gi[:, 128:256], gi[:, 256:]
    hr, hz, hn = gh[:, :128], gh[:, 128:256], gh[:, 256:]
    r = jax.nn.sigmoid(ir + hr)
    z = jax.nn.sigmoid(iz + hz)
    n = jnp.tanh(inn + r * hn)
    h_ref[...] = (1.0 - z) * n + z * prev
    agg = jnp.where(cnt > 0.0, a_ref[...] + be_ref[...] + m_ref[...], 0.0)
    agg_ref[...] = agg
    ea = jax.nn.relu(agg)
    an_ref[...] = jnp.dot(ea, wa_ref[...], preferred_element_type=jnp.float32)
    bn_ref[...] = jnp.dot(ea, wb_ref[...], preferred_element_type=jnp.float32)


def _tc_b(hsum, prev, cnt, A, M, WlT, WrT, bl3, WihT, WhhT, bih, bhh, be,
          WA, WB, n_pad):
    grid = n_pad // _BLK
    blk = lambda i: (i, 0)
    full = lambda i: (0, 0)
    return pl.pallas_call(
        _tc_b_body,
        grid=(grid,),
        in_specs=[
            pl.BlockSpec((_BLK, 128), blk),
            pl.BlockSpec((_BLK, 128), blk),
            pl.BlockSpec((_BLK, 1), blk),
            pl.BlockSpec((_BLK, 128), blk),
            pl.BlockSpec((_BLK, 128), blk),
            pl.BlockSpec((128, (_MAX_DEG + 1) * 128), full),
            pl.BlockSpec((128, (_MAX_DEG + 1) * 128), full),
            pl.BlockSpec((1, _MAX_DEG + 1, 128), lambda i: (0, 0, 0)),
            pl.BlockSpec((128, 384), full),
            pl.BlockSpec((128, 384), full),
            pl.BlockSpec((1, 384), full),
            pl.BlockSpec((1, 384), full),
            pl.BlockSpec((1, 128), full),
            pl.BlockSpec((128, 128), full),
            pl.BlockSpec((128, 128), full),
        ],
        out_specs=[pl.BlockSpec((_BLK, 128), blk)] * 5,
        out_shape=[jax.ShapeDtypeStruct((n_pad, 128), jnp.float32)] * 5,
    )(hsum, prev, cnt, A, M, WlT, WrT, bl3, WihT, WhhT, bih, bhh, be, WA, WB)


def _tc_d_body(s1_ref, agg2_ref, cnt_ref, oh_ref, wp1t_ref, bp1_ref,
               wp2t_ref, bp2_ref, g_ref, acc_ref):
    i = pl.program_id(0)

    @pl.when(i == 0)
    def _():
        acc_ref[...] = jnp.zeros_like(acc_ref)

    s2 = jax.nn.relu(agg2_ref[...]) * cnt_ref[...]
    oh = oh_ref[...]
    p1 = jnp.dot(oh.T, s1_ref[...], preferred_element_type=jnp.float32)
    p2 = jnp.dot(oh.T, s2, preferred_element_type=jnp.float32)
    acc_ref[...] += jnp.concatenate([p1, p2], axis=1)

    @pl.when(i == pl.num_programs(0) - 1)
    def _():
        gsum = acc_ref[...]
        hid = jax.nn.relu(
            jnp.dot(gsum, wp1t_ref[...], preferred_element_type=jnp.float32)
            + bp1_ref[...])
        g_ref[...] = jnp.dot(hid, wp2t_ref[...],
                             preferred_element_type=jnp.float32) + bp2_ref[...]


def _tc_d(s1, agg2, cnt, oh, Wp1T, bp1, Wp2T, bp2, n_pad):
    grid = n_pad // _BLK
    blk = lambda i: (i, 0)
    full = lambda i: (0, 0)
    return pl.pallas_call(
        _tc_d_body,
        grid=(grid,),
        in_specs=[
            pl.BlockSpec((_BLK, 128), blk),
            pl.BlockSpec((_BLK, 128), blk),
            pl.BlockSpec((_BLK, 1), blk),
            pl.BlockSpec((_BLK, 16), blk),
            pl.BlockSpec((256, 32), full),
            pl.BlockSpec((1, 32), full),
            pl.BlockSpec((32, 1), full),
            pl.BlockSpec((1, 1), full),
        ],
        out_specs=pl.BlockSpec((16, 1), full),
        out_shape=jax.ShapeDtypeStruct((16, 1), jnp.float32),
        scratch_shapes=[pltpu.VMEM((16, 256), jnp.float32)],
        compiler_params=pltpu.CompilerParams(
            dimension_semantics=("arbitrary",)),
    )(s1, agg2, cnt, oh, Wp1T, bp1, Wp2T, bp2)


def _emb_edge_body(agg2_ref, out_ref):
    i = pl.program_id(0)
    out_ref[...] = jnp.where(i < 5, agg2_ref[...], 0.0)


def _emb_edge(agg2, E):
    blk_rows = 2000
    grid = E // blk_rows
    return pl.pallas_call(
        _emb_edge_body,
        grid=(grid,),
        in_specs=[pl.BlockSpec((blk_rows, 128),
                               lambda i: (jnp.minimum(i, 4), 0))],
        out_specs=pl.BlockSpec((blk_rows, 128), lambda i: (i, 0)),
        out_shape=jax.ShapeDtypeStruct((E, 128), jnp.float32),
    )(agg2)


_CH = 256
_SUB = 128


def _sc_segsum(src, row2, col2, z2d, z1d, n_pad, E, with_count):
    nchunk = E // _CH
    tpw = (nchunk + 31) // 32
    nrt = n_pad // 16
    k_sub = _CH // _SUB
    mesh = plsc.VectorSubcoreMesh(core_axis_name="c", subcore_axis_name="s")
    out_type = [jax.ShapeDtypeStruct((2, n_pad, 128), jnp.float32)]
    if with_count:
        out_type.append(jax.ShapeDtypeStruct((2, n_pad // 128, 128),
                                             jnp.float32))
    scratch = [
        pltpu.VMEM((k_sub, _SUB), jnp.int32),
        pltpu.VMEM((k_sub, _SUB), jnp.int32),
        pltpu.VMEM((_CH, 128), jnp.float32),
        pltpu.VMEM((_SUB,), jnp.float32),
        pltpu.VMEM((1024,), jnp.float32),
        pltpu.VMEM((8, 128), jnp.float32),
        pltpu.VMEM_SHARED((n_pad, 128), jnp.float32),
        pltpu.VMEM_SHARED((n_pad,), jnp.float32),
        pltpu.SemaphoreType.DMA,
    ]

    @functools.partial(pl.kernel, out_type=out_type, mesh=mesh,
                       scratch_types=scratch)
    def body(src_h, row_h, col_h, z2_h, z1_h, *rest):
        if with_count:
            psum_h, pcnt_h = rest[0], rest[1]
            scr = rest[2:]
        else:
            psum_h = rest[0]
            scr = rest[1:]
        idxr, idxc, rows, ones, cnt1, cnt2, acc_sh, cnt_sh, sem = scr
        cc = lax.axis_index("c")
        ss = lax.axis_index("s")
        wid = cc * 16 + ss
        pltpu.sync_copy(z2_h.at[pl.ds(ss * nrt, nrt)],
                        acc_sh.at[pl.ds(ss * nrt, nrt)])
        if with_count:
            pltpu.sync_copy(z1_h.at[pl.ds(ss * nrt, nrt)],
                            cnt_sh.at[pl.ds(ss * nrt, nrt)])
            for k in range(_SUB // 16):
                ones[pl.ds(k * 16, 16)] = jnp.ones((16,), jnp.float32)
        plsc.subcore_barrier()

        def chunk(t, carry):
            j = t * 32 + wid

            @pl.when(j < nchunk)
            def _do():
                r0 = j * (_CH // 128)
                pltpu.sync_copy(row_h.at[pl.ds(r0, k_sub)], idxr)
                pltpu.sync_copy(col_h.at[pl.ds(r0, k_sub)], idxc)
                cps = [pltpu.async_copy(src_h.at[idxr.at[k]],
                                        rows.at[pl.ds(k * _SUB, _SUB)], sem)
                       for k in range(k_sub)]
                for cp in cps:
                    cp.wait()
                for k in range(k_sub):
                    pltpu.sync_copy(rows.at[pl.ds(k * _SUB, _SUB)],
                                    acc_sh.at[idxc.at[k]], add=True)
                    if with_count:
                        pltpu.sync_copy(ones, cnt_sh.at[idxc.at[k]],
                                        add=True)
            return carry

        lax.fori_loop(0, tpw, chunk, None)
        plsc.subcore_barrier()
        pltpu.sync_copy(acc_sh.at[pl.ds(ss * nrt, nrt)],
                        psum_h.at[cc].at[pl.ds(ss * nrt, nrt)])
        if with_count:
            @pl.when(ss < n_pad // 1024)
            def _wb():
                pltpu.sync_copy(cnt_sh.at[pl.ds(ss * 1024, 1024)], cnt1)
                for k in range(64):
                    cnt2[k // 8, pl.ds((k % 8) * 16, 16)] = \
                        cnt1[pl.ds(k * 16, 16)]
                pltpu.sync_copy(cnt2, pcnt_h.at[cc].at[pl.ds(ss * 8, 8)])

    res = body(src, row2, col2, z2d, z1d)
    if with_count:
        return res[0], res[1]
    return res[0] if isinstance(res, (list, tuple)) else res


def kernel(x, edge_attr, edge_index, batch, W0, b0, Wl, bl, Wr, We, be,
           Wih, Whh, bih, bhh, Wp1, bp1, Wp2, bp2):
    N, D = x.shape
    E = edge_attr.shape[0]
    n_pad = ((N + _BLK - 1) // _BLK) * _BLK
    row, col = edge_index[0], edge_index[1]

    ED = We.shape[0]
    W0T = W0.T
    WA = (We[:, :ED] - We[:, ED:]).T
    WB = We[:, ED:].T
    WlT = jnp.transpose(Wl, (2, 0, 1)).reshape(D, (_MAX_DEG + 1) * D)
    WrT = jnp.transpose(Wr, (2, 0, 1)).reshape(D, (_MAX_DEG + 1) * D)
    bl3 = bl.reshape(1, _MAX_DEG + 1, D)
    WihT, WhhT = Wih.T, Whh.T
    Wp1T, Wp2T = Wp1.T, Wp2.T
    b0r = b0.reshape(1, D)
    ber = be.reshape(1, D)
    bihr, bhhr = bih.reshape(1, 3 * D), bhh.reshape(1, 3 * D)
    bp1r, bp2r = bp1.reshape(1, 32), bp2.reshape(1, 1)

    pad = n_pad - N
    xp = jnp.pad(x, ((0, pad), (0, 0)))
    eap = jnp.pad(edge_attr[:N], ((0, pad), (0, 0)))
    ohp = jnp.pad(
        (batch[:, None] == jnp.arange(_NUM_GRAPHS)[None, :]).astype(
            jnp.float32), ((0, pad), (0, 0)))

    out0, A1, B1 = _tc_a(xp, eap, W0T, b0r, WA, WB, n_pad)

    row2 = row.reshape(E // 128, 128)
    col2 = col.reshape(E // 128, 128)
    z2d = jnp.zeros((n_pad, 128), jnp.float32)
    z1d = jnp.zeros((n_pad,), jnp.float32)

    psum1, pcnt = _sc_segsum(out0, row2, col2, z2d, z1d, n_pad, E, True)
    cnt = (pcnt[0] + pcnt[1]).reshape(n_pad, 1)
    hsum1 = psum1[0] + psum1[1]
    M1 = jax.ops.segment_max(B1[row], col, num_segments=n_pad)
    M1 = jnp.where(jnp.isfinite(M1), M1, 0.0)

    h1, _m1, _agg1, A2, B2 = _tc_b(hsum1, out0, cnt, A1, M1, WlT, WrT, bl3,
                                   WihT, WhhT, bihr, bhhr, ber, WA, WB, n_pad)

    psum2 = _sc_segsum(h1, row2, col2, z2d, z1d, n_pad, E, False)
    hsum2 = psum2[0] + psum2[1]
    M2 = jax.ops.segment_max(B2[row], col, num_segments=n_pad)
    M2 = jnp.where(jnp.isfinite(M2), M2, 0.0)

    h2, m2, agg2, _A3, _B3 = _tc_b(hsum2, h1, cnt, A2, M2, WlT, WrT, bl3,
                                   WihT, WhhT, bihr, bhhr, ber, WA, WB, n_pad)

    psum3 = _sc_segsum(h2, row2, col2, z2d, z1d, n_pad, E, False)
    s1 = psum3[0] + psum3[1]

    g = _tc_d(s1, agg2, cnt, ohp, Wp1T, bp1r, Wp2T, bp2r, n_pad)

    emb_node = m2[:N]
    emb_edge = _emb_edge(agg2[:N], E)
    return (g, emb_node, emb_edge)
